# Initial kernel scaffold; baseline (speedup 1.0000x reference)
import jax
import jax.numpy as jnp
from jax.experimental import pallas as pl


def kernel(idx, v):
    raise NotImplementedError("write your pallas kernel here")


# speedup vs baseline: 1.2717x; 1.2717x over previous
import functools

import jax
import jax.numpy as jnp
from jax import lax
from jax.experimental import pallas as pl
from jax.experimental.pallas import tpu as pltpu
from jax.experimental.pallas import tpu_sc as plsc

_INFO = plsc.get_sparse_core_info()
_NC, _NS = _INFO.num_cores, _INFO.num_subcores
_NW = _NC * _NS


def _make_gather(B: int):
    assert B % _NW == 0
    b_per_w = B // _NW
    assert b_per_w % 8 == 0
    mesh = plsc.VectorSubcoreMesh(core_axis_name="c", subcore_axis_name="s")

    @functools.partial(
        pl.kernel,
        mesh=mesh,
        out_type=jax.ShapeDtypeStruct((B,), jnp.float32),
        scratch_types=[
            pltpu.VMEM((b_per_w,), jnp.int32),
            pltpu.VMEM((b_per_w,), jnp.float32),
            pltpu.SemaphoreType.DMA,
        ],
    )
    def gather_kernel(idx_hbm, table_hbm, out_hbm, idx_v, rows_v, sem):
        wid = lax.axis_index("s") * _NC + lax.axis_index("c")
        base = wid * b_per_w
        pltpu.sync_copy(idx_hbm.at[pl.ds(base, b_per_w)], idx_v)
        pltpu.async_copy(table_hbm.at[idx_v], rows_v, sem).wait()
        pltpu.sync_copy(rows_v, out_hbm.at[pl.ds(base, b_per_w)])

    return gather_kernel


@jax.jit
def kernel(idx, v):
    n, m = idx.shape
    flat = jnp.reshape(idx.astype(jnp.int32), (n * m,))
    out = _make_gather(n * m)(flat, v)
    return jnp.reshape(out, (n, m))


# speedup vs baseline: 1.6871x; 1.3266x over previous
import functools

import jax
import jax.numpy as jnp
from jax import lax
from jax.experimental import pallas as pl
from jax.experimental.pallas import tpu as pltpu
from jax.experimental.pallas import tpu_sc as plsc

_INFO = plsc.get_sparse_core_info()
_NC, _NS = _INFO.num_cores, _INFO.num_subcores
_NW = _NC * _NS


def _make_gather(n: int, m: int):
    assert n % _NW == 0
    rows_per_w = n // _NW
    mesh = plsc.VectorSubcoreMesh(core_axis_name="c", subcore_axis_name="s")

    b_per_w = rows_per_w * m

    @functools.partial(
        pl.kernel,
        mesh=mesh,
        out_type=jax.ShapeDtypeStruct((n, m), jnp.float32),
        scratch_types=[
            pltpu.VMEM((rows_per_w, m), jnp.int32),
            pltpu.VMEM((rows_per_w, m), jnp.float32),
            pltpu.SemaphoreType.DMA,
        ],
    )
    def gather_kernel(idx_hbm, table_hbm, out_hbm, idx_v, rows_v, sem):
        wid = lax.axis_index("s") * _NC + lax.axis_index("c")
        base = wid * rows_per_w
        pltpu.sync_copy(idx_hbm.at[pl.ds(base, rows_per_w), :], idx_v)

        def fire(r, carry):
            pltpu.make_async_copy(
                table_hbm.at[idx_v.at[r]], rows_v.at[r], sem
            ).start()
            return carry

        lax.fori_loop(0, rows_per_w, fire, 0)

        def drain(r, carry):
            pltpu.make_async_copy(
                table_hbm.at[idx_v.at[r]], rows_v.at[r], sem
            ).wait()
            return carry

        lax.fori_loop(0, rows_per_w, drain, 0)
        pltpu.sync_copy(rows_v, out_hbm.at[pl.ds(base, rows_per_w), :])

    return gather_kernel


@jax.jit
def kernel(idx, v):
    n, m = idx.shape
    return _make_gather(n, m)(idx.astype(jnp.int32), v)


# speedup vs baseline: 2.6391x; 1.5643x over previous
import functools

import jax
import jax.numpy as jnp
from jax import lax
from jax.experimental import pallas as pl
from jax.experimental.pallas import tpu as pltpu
from jax.experimental.pallas import tpu_sc as plsc

_INFO = plsc.get_sparse_core_info()
_NC, _NS = _INFO.num_cores, _INFO.num_subcores
_NW = _NC * _NS


def _make_gather(n: int, m: int, v_len: int):
    assert n % _NW == 0
    rows_per_w = n // _NW
    mesh = plsc.VectorSubcoreMesh(core_axis_name="c", subcore_axis_name="s")

    chunk = 128
    n_chunks = rows_per_w // chunk
    assert rows_per_w % chunk == 0

    @functools.partial(
        pl.kernel,
        mesh=mesh,
        out_type=jax.ShapeDtypeStruct((n, m), jnp.float32),
        scratch_types=[
            pltpu.VMEM((chunk, m), jnp.int32),
            pltpu.VMEM((chunk, m), jnp.float32),
            pltpu.VMEM_SHARED((v_len,), jnp.float32),
            pltpu.SemaphoreType.DMA,
        ],
    )
    def gather_kernel(idx_hbm, table_hbm, out_hbm, idx_v, rows_v, tbl_s, sem):
        sid = lax.axis_index("s")
        wid = sid * _NC + lax.axis_index("c")
        base = wid * rows_per_w

        @pl.when(sid == 0)
        def _():
            pltpu.sync_copy(table_hbm, tbl_s)

        plsc.subcore_barrier()

        def do_chunk(c, carry):
            cbase = base + c * chunk
            pltpu.sync_copy(idx_hbm.at[pl.ds(cbase, chunk), :], idx_v)

            def fire(r, cy):
                pltpu.make_async_copy(
                    tbl_s.at[idx_v.at[r]], rows_v.at[r], sem
                ).start()
                return cy

            lax.fori_loop(0, chunk, fire, 0)

            def drain(r, cy):
                pltpu.make_async_copy(
                    tbl_s.at[idx_v.at[r]], rows_v.at[r], sem
                ).wait()
                return cy

            lax.fori_loop(0, chunk, drain, 0)
            pltpu.sync_copy(rows_v, out_hbm.at[pl.ds(cbase, chunk), :])
            return carry

        lax.fori_loop(0, n_chunks, do_chunk, 0)

    return gather_kernel


@jax.jit
def kernel(idx, v):
    n, m = idx.shape
    return _make_gather(n, m, v.shape[0])(idx.astype(jnp.int32), v)
